# baseline (speedup 1.0000x reference)
import functools

import jax
import jax.numpy as jnp
from jax.experimental import pallas as pl


def _dot(a, b, trans_b=False):
    dims = (((1,), (1 if trans_b else 0,)), ((), ()))
    return jax.lax.dot_general(a, b, dims, preferred_element_type=jnp.float32)


def _qkv_body(x_ref, w_ref, o_ref, *, c):
    y = _dot(x_ref[...], w_ref[...])
    q = y[:, :c]
    k = y[:, c:2 * c]
    v = y[:, 2 * c:]
    qn = q / jnp.maximum(jnp.sqrt(jnp.sum(q * q, axis=-1, keepdims=True)), 1e-12)
    kn = k / jnp.maximum(jnp.sqrt(jnp.sum(k * k, axis=-1, keepdims=True)), 1e-12)
    o_ref[...] = jnp.concatenate([qn, kn, v], axis=1)


def _attn_body(q_ref, k_ref, v_ref, rel_ref, o_ref, *, bt, off, scale):
    q = q_ref[0, 0]
    k = k_ref[0, 0]
    v = v_ref[0, 0]
    s = _dot(q, k, trans_b=True)
    s = (s + rel_ref[...]) * scale
    t = pl.program_id(1)
    rows = t * bt + jax.lax.broadcasted_iota(jnp.int32, s.shape, 0)
    cols = jax.lax.broadcasted_iota(jnp.int32, s.shape, 1)
    s = jnp.where(cols <= rows + off, s, -1e30)
    m = jnp.max(s, axis=-1, keepdims=True)
    p = jnp.exp(s - m)
    l = jnp.sum(p, axis=-1, keepdims=True)
    o_ref[0, 0] = _dot(p, v) / l


def _proj_body(x_ref, w_ref, b_ref, o_ref):
    o_ref[...] = _dot(x_ref[...], w_ref[...]) + b_ref[...]


def kernel(batch_file_idxs, relative_positions, x, xl_memory, Wq, Wk, Wv, Wp,
           bp, gate_bias):
    del batch_file_idxs, gate_bias
    B, T, C = x.shape
    TXL = xl_memory.shape[1]
    J = T + TXL
    H = 16
    D = C // H
    scale = float(D) ** -0.5

    BT1 = min(512, B * T)
    x2 = x.reshape(B * T, C)
    w_qkv = jnp.concatenate([Wq, Wk, Wv], axis=0).T
    qkv = pl.pallas_call(
        functools.partial(_qkv_body, c=C),
        grid=(B * T // BT1,),
        in_specs=[
            pl.BlockSpec((BT1, C), lambda i: (i, 0)),
            pl.BlockSpec((C, 3 * C), lambda i: (0, 0)),
        ],
        out_specs=pl.BlockSpec((BT1, 3 * C), lambda i: (i, 0)),
        out_shape=jax.ShapeDtypeStruct((B * T, 3 * C), jnp.float32),
    )(x2, w_qkv)
    qn = qkv[:, :C].reshape(B, T, C)
    kn = qkv[:, C:2 * C].reshape(B, T, C)
    v = qkv[:, 2 * C:].reshape(B, T, C)

    k_cat = jnp.concatenate([xl_memory[:, :, 0, :], kn], axis=1)
    v_cat = jnp.concatenate([xl_memory[:, :, 1, :], v], axis=1)
    qh = qn.reshape(B, T, H, D).transpose(0, 2, 1, 3)
    kh = k_cat.reshape(B, J, H, D).transpose(0, 2, 1, 3)
    vh = v_cat.reshape(B, J, H, D).transpose(0, 2, 1, 3)
    rel = relative_positions.reshape(relative_positions.shape[-2],
                                     relative_positions.shape[-1])[-T:, -J:]

    BT2 = min(256, T)
    wv = pl.pallas_call(
        functools.partial(_attn_body, bt=BT2, off=J - T, scale=scale),
        grid=(B, T // BT2, H),
        in_specs=[
            pl.BlockSpec((1, 1, BT2, D), lambda b, t, h: (b, h, t, 0)),
            pl.BlockSpec((1, 1, J, D), lambda b, t, h: (b, h, 0, 0)),
            pl.BlockSpec((1, 1, J, D), lambda b, t, h: (b, h, 0, 0)),
            pl.BlockSpec((BT2, J), lambda b, t, h: (t, 0)),
        ],
        out_specs=pl.BlockSpec((1, 1, BT2, D), lambda b, t, h: (b, h, t, 0)),
        out_shape=jax.ShapeDtypeStruct((B, H, T, D), jnp.float32),
    )(qh, kh, vh, rel)
    wv2 = wv.transpose(0, 2, 1, 3).reshape(B * T, C)

    out = pl.pallas_call(
        _proj_body,
        grid=(B * T // BT1,),
        in_specs=[
            pl.BlockSpec((BT1, C), lambda i: (i, 0)),
            pl.BlockSpec((C, C), lambda i: (0, 0)),
            pl.BlockSpec((1, C), lambda i: (0, 0)),
        ],
        out_specs=pl.BlockSpec((BT1, C), lambda i: (i, 0)),
        out_shape=jax.ShapeDtypeStruct((B * T, C), jnp.float32),
    )(wv2, Wp.T, bp.reshape(1, C))
    out = out.reshape(B, T, C)

    new_xl_memory = jnp.stack([kn, v], axis=2)
    return (out, new_xl_memory)
